# baseline (speedup 1.0000x reference)
import functools

import jax
import jax.numpy as jnp
from jax import lax
from jax.experimental import pallas as pl
from jax.experimental.pallas import tpu as pltpu
from jax.experimental.pallas import tpu_sc as plsc

NC = 2
NS = 16
LANES = 16


def _sc_mesh():
    return plsc.VectorSubcoreMesh(
        core_axis_name="c", subcore_axis_name="s", num_cores=NC, num_subcores=NS
    )


def _pick_chunk(per_worker):
    for ch in range(128, 0, -8):
        if per_worker % ch == 0:
            return ch
    raise ValueError(f"no valid chunk for {per_worker}")


def _deg_histogram(col, n):
    e = col.shape[0]
    npad = -(-n // (NS * LANES)) * (NS * LANES)
    slab = npad // NS
    ew = e // (NC * NS)
    ch = _pick_chunk(ew)
    k_sub = 5 if (ew // ch) % 5 == 0 else 1
    iters = ew // (ch * k_sub)
    assert iters % 2 == 1 and iters >= 3, (ew, ch, k_sub)

    @functools.partial(
        pl.kernel,
        out_type=jax.ShapeDtypeStruct((NC, npad), jnp.float32),
        mesh=_sc_mesh(),
        scratch_types=[
            pltpu.VMEM((2, k_sub, ch), jnp.int32),
            pltpu.VMEM((ch,), jnp.float32),
            pltpu.VMEM((slab,), jnp.float32),
            pltpu.VMEM_SHARED((npad,), jnp.float32),
            pltpu.SemaphoreType.DMA,
            pltpu.SemaphoreType.DMA,
            pltpu.SemaphoreType.DMA,
            pltpu.SemaphoreType.DMA,
        ],
    )
    def k(col_hbm, out_hbm, cidx, ones, zbuf, acc, si0, si1, ss0, ss1):
        c = lax.axis_index("c")
        s = lax.axis_index("s")
        w = c * NS + s
        base = w * ew
        sem_i = (si0, si1)
        sem_s = (ss0, ss1)

        def fire_idx(b, u):
            off = base + b * (ch * k_sub)
            for j in range(k_sub):
                pltpu.async_copy(col_hbm.at[pl.ds(off + j * ch, ch)],
                                 cidx.at[u, j], sem_i[u])

        def drain_idx(u):
            for j in range(k_sub):
                pltpu.make_async_copy(col_hbm.at[pl.ds(base, ch)],
                                      cidx.at[u, j], sem_i[u]).wait()

        def fire_scat(u):
            for j in range(k_sub):
                pltpu.async_copy(ones, acc.at[cidx.at[u, j]], sem_s[u],
                                 add=True)

        def drain_scat(u):
            for j in range(k_sub):
                pltpu.make_async_copy(ones, acc.at[cidx.at[u, j]],
                                      sem_s[u]).wait()

        fire_idx(0, 0)
        fire_idx(1, 1)

        def fill(i, _):
            zbuf[pl.ds(i * LANES, LANES)] = jnp.zeros((LANES,), jnp.float32)
            return _

        lax.fori_loop(0, slab // LANES, fill, 0)
        for j in range(ch // LANES):
            ones[pl.ds(j * LANES, LANES)] = jnp.ones((LANES,), jnp.float32)
        pltpu.sync_copy(zbuf, acc.at[pl.ds(s * slab, slab)])
        plsc.subcore_barrier()

        drain_idx(0)
        fire_scat(0)

        def pair(t, _):
            b1 = 1 + 2 * t
            b2 = 2 + 2 * t
            drain_idx(1)
            fire_scat(1)
            drain_scat(0)

            @pl.when(b1 + 1 < iters)
            def _p1():
                fire_idx(b1 + 1, 0)
                drain_idx(0)

            @pl.when(b1 + 1 < iters)
            def _s2():
                fire_scat(0)
            drain_scat(1)

            @pl.when(b2 + 1 < iters)
            def _p2():
                fire_idx(b2 + 1, 1)

            return _

        lax.fori_loop(0, (iters - 1) // 2, pair, 0)
        drain_scat(0)
        plsc.subcore_barrier()
        pltpu.sync_copy(acc.at[pl.ds(s * slab, slab)],
                        out_hbm.at[c, pl.ds(s * slab, slab)])

    return k(col)


def _edge_scatter(hs, row, col):
    n, d = hs.shape
    e = row.shape[0]
    npad = -(-n // (NS * 128)) * (NS * 128)
    slab = npad // NS
    ew = e // (NC * NS)
    sub = _pick_chunk(ew)
    g_total = ew // sub
    t_body = (g_total - 3) // 6
    r_tail = g_total - 6 * t_body
    assert g_total >= 6, (ew, sub)
    nfull = n // slab
    rem = n - nfull * slab
    zr = 8
    for cand in range(min(slab, 64), 0, -8):
        if slab % cand == 0 and (rem == 0 or (slab - rem) % cand == 0):
            zr = cand
            break

    @functools.partial(
        pl.kernel,
        out_type=jax.ShapeDtypeStruct((NC, npad, d), jnp.float32),
        mesh=_sc_mesh(),
        scratch_types=[
            pltpu.VMEM((6, 2, sub), jnp.int32),
            pltpu.VMEM((3, sub, d), jnp.float32),
            pltpu.VMEM((zr, d), jnp.float32),
            pltpu.VMEM_SHARED((npad, d), jnp.float32),
            pltpu.SemaphoreType.DMA,
            pltpu.SemaphoreType.DMA,
            pltpu.SemaphoreType.DMA,
            pltpu.SemaphoreType.DMA,
            pltpu.SemaphoreType.DMA,
        ],
    )
    def k(hs_hbm, row_hbm, col_hbm, out_hbm, eidx, rows, zbuf, acc,
          sem_i, sem_g0, sem_g1, sem_g2, sem_s):
        c = lax.axis_index("c")
        s = lax.axis_index("s")
        w = c * NS + s
        base = w * ew
        sem_g = (sem_g0, sem_g1, sem_g2)

        def fire_eidx(g, ib):
            off = base + g * sub
            pltpu.async_copy(row_hbm.at[pl.ds(off, sub)], eidx.at[ib, 0], sem_i)
            pltpu.async_copy(col_hbm.at[pl.ds(off, sub)], eidx.at[ib, 1], sem_i)

        def drain_eidx(ib):
            pltpu.make_async_copy(row_hbm.at[pl.ds(base, sub)],
                                  eidx.at[ib, 0], sem_i).wait()
            pltpu.make_async_copy(col_hbm.at[pl.ds(base, sub)],
                                  eidx.at[ib, 1], sem_i).wait()

        def fire_gather(rb, ib):
            pltpu.async_copy(hs_hbm.at[eidx.at[ib, 0]], rows.at[rb], sem_g[rb])

        def drain_gather(rb, ib):
            pltpu.make_async_copy(hs_hbm.at[eidx.at[ib, 0]], rows.at[rb],
                                  sem_g[rb]).wait()

        def fire_scat(rb, ib):
            pltpu.async_copy(rows.at[rb], acc.at[eidx.at[ib, 1]], sem_s,
                             add=True)

        def drain_scat(rb, ib):
            pltpu.make_async_copy(rows.at[rb], acc.at[eidx.at[ib, 1]],
                                  sem_s).wait()

        for g in range(3):
            fire_eidx(g, g)

        def fill(i, _):
            for j in range(d // LANES):
                zbuf[i, pl.ds(j * LANES, LANES)] = jnp.zeros((LANES,), jnp.float32)
            return _

        lax.fori_loop(0, zr, fill, 0)
        for g in range(3):
            drain_eidx(g)
        for g in range(3):
            fire_gather(g, g)

        @pl.when(jnp.logical_and(c == 0, s < nfull))
        def _seed_full():
            pltpu.sync_copy(hs_hbm.at[pl.ds(s * slab, slab)],
                            acc.at[pl.ds(s * slab, slab)])

        if rem:
            @pl.when(jnp.logical_and(c == 0, s == nfull))
            def _seed_rem():
                pltpu.sync_copy(hs_hbm.at[pl.ds(nfull * slab, rem)],
                                acc.at[pl.ds(nfull * slab, rem)])
                for t in range((slab - rem) // zr):
                    pltpu.sync_copy(
                        zbuf, acc.at[pl.ds(nfull * slab + rem + t * zr, zr)])

        @pl.when(jnp.logical_or(c != 0, s > nfull if rem else s >= nfull))
        def _zero():
            for t in range(slab // zr):
                pltpu.sync_copy(zbuf, acc.at[pl.ds(s * slab + t * zr, zr)])

        plsc.subcore_barrier()

        def slot(g, u):
            rb = u % 3
            ib = u % 6
            ib2 = (u + 3) % 6
            fire_eidx(g + 3, ib2)
            drain_gather(rb, ib)
            fire_scat(rb, ib)
            drain_scat(rb, ib)
            drain_eidx(ib2)
            fire_gather(rb, ib2)

        def body(t, _):
            g0 = 6 * t
            for u in range(6):
                slot(g0 + u, u)
            return _

        lax.fori_loop(0, t_body, body, 0)

        g0 = 6 * t_body
        for u in range(r_tail):
            rb = u % 3
            ib = u % 6
            if u + 3 < r_tail:
                fire_eidx(g0 + u + 3, (u + 3) % 6)
            drain_gather(rb, ib)
            fire_scat(rb, ib)
            drain_scat(rb, ib)
            if u + 3 < r_tail:
                drain_eidx((u + 3) % 6)
                fire_gather(rb, (u + 3) % 6)

        plsc.subcore_barrier()
        pltpu.sync_copy(acc.at[pl.ds(s * slab, slab)],
                        out_hbm.at[c, pl.ds(s * slab, slab)])

    return k(hs, row, col)


def _dis_col(deg_ref, n):
    deg = deg_ref[0, :n] + deg_ref[1, :n] + 1.0
    return lax.rsqrt(deg)[:, None]


def _mm_pre(x, W, degp):
    n, d = x.shape

    def body(x_ref, w_ref, deg_ref, o_ref):
        h = jnp.dot(x_ref[...], w_ref[...], preferred_element_type=jnp.float32)
        o_ref[...] = h * _dis_col(deg_ref, n)

    return pl.pallas_call(
        body, out_shape=jax.ShapeDtypeStruct((n, d), jnp.float32)
    )(x, W, degp)


def _mm_mid(s1, degp, W2, b1, n, d):
    def body(s1_ref, deg_ref, w_ref, b_ref, o_ref):
        dis = _dis_col(deg_ref, n)
        pre = dis * (s1_ref[0, :n] + s1_ref[1, :n]) + b_ref[...][None, :]
        a = jnp.maximum(pre, 0.0)
        o_ref[...] = jnp.dot(a, w_ref[...], preferred_element_type=jnp.float32) * dis

    return pl.pallas_call(
        body, out_shape=jax.ShapeDtypeStruct((n, d), jnp.float32)
    )(s1, degp, W2, b1)


def _mm_post(s2, degp, b2, n, d):
    def body(s2_ref, deg_ref, b_ref, o_ref):
        dis = _dis_col(deg_ref, n)
        o_ref[...] = (dis * (s2_ref[0, :n] + s2_ref[1, :n])
                      + b_ref[...][None, :])

    return pl.pallas_call(
        body, out_shape=jax.ShapeDtypeStruct((n, d), jnp.float32)
    )(s2, degp, b2)


def kernel(x, edge_index, W1, b1, W2, b2):
    n, d = x.shape
    row = edge_index[0]
    col = edge_index[1]
    degp = _deg_histogram(col, n)
    hs1 = _mm_pre(x, W1, degp)
    s1 = _edge_scatter(hs1, row, col)
    hs2 = _mm_mid(s1, degp, W2, b1, n, d)
    s2 = _edge_scatter(hs2, row, col)
    return _mm_post(s2, degp, b2, n, d)


# speedup vs baseline: 1.0180x; 1.0180x over previous
import functools

import jax
import jax.numpy as jnp
from jax import lax
from jax.experimental import pallas as pl
from jax.experimental.pallas import tpu as pltpu
from jax.experimental.pallas import tpu_sc as plsc

NC = 2
NS = 16
LANES = 16


def _sc_mesh():
    return plsc.VectorSubcoreMesh(
        core_axis_name="c", subcore_axis_name="s", num_cores=NC, num_subcores=NS
    )


def _pick_chunk(per_worker):
    for ch in range(128, 0, -8):
        if per_worker % ch == 0:
            return ch
    raise ValueError(f"no valid chunk for {per_worker}")


def _deg_histogram(col, n):
    e = col.shape[0]
    npad = -(-n // (NS * LANES)) * (NS * LANES)
    slab = npad // NS
    ew = e // (NC * NS)
    ch = _pick_chunk(ew)
    k_sub = 5 if (ew // ch) % 5 == 0 else 1
    iters = ew // (ch * k_sub)
    assert iters % 2 == 1 and iters >= 3, (ew, ch, k_sub)

    @functools.partial(
        pl.kernel,
        out_type=jax.ShapeDtypeStruct((NC, npad), jnp.float32),
        mesh=_sc_mesh(),
        scratch_types=[
            pltpu.VMEM((2, k_sub, ch), jnp.int32),
            pltpu.VMEM((ch,), jnp.float32),
            pltpu.VMEM((slab,), jnp.float32),
            pltpu.VMEM_SHARED((npad,), jnp.float32),
            pltpu.SemaphoreType.DMA,
            pltpu.SemaphoreType.DMA,
            pltpu.SemaphoreType.DMA,
            pltpu.SemaphoreType.DMA,
        ],
    )
    def k(col_hbm, out_hbm, cidx, ones, zbuf, acc, si0, si1, ss0, ss1):
        c = lax.axis_index("c")
        s = lax.axis_index("s")
        w = c * NS + s
        base = w * ew
        sem_i = (si0, si1)
        sem_s = (ss0, ss1)

        def fire_idx(b, u):
            off = base + b * (ch * k_sub)
            for j in range(k_sub):
                pltpu.async_copy(col_hbm.at[pl.ds(off + j * ch, ch)],
                                 cidx.at[u, j], sem_i[u])

        def drain_idx(u):
            for j in range(k_sub):
                pltpu.make_async_copy(col_hbm.at[pl.ds(base, ch)],
                                      cidx.at[u, j], sem_i[u]).wait()

        def fire_scat(u):
            for j in range(k_sub):
                pltpu.async_copy(ones, acc.at[cidx.at[u, j]], sem_s[u],
                                 add=True)

        def drain_scat(u):
            for j in range(k_sub):
                pltpu.make_async_copy(ones, acc.at[cidx.at[u, j]],
                                      sem_s[u]).wait()

        fire_idx(0, 0)
        fire_idx(1, 1)

        def fill(i, _):
            zbuf[pl.ds(i * LANES, LANES)] = jnp.zeros((LANES,), jnp.float32)
            return _

        lax.fori_loop(0, slab // LANES, fill, 0)
        for j in range(ch // LANES):
            ones[pl.ds(j * LANES, LANES)] = jnp.ones((LANES,), jnp.float32)
        pltpu.sync_copy(zbuf, acc.at[pl.ds(s * slab, slab)])
        plsc.subcore_barrier()

        drain_idx(0)
        fire_scat(0)

        def pair(t, _):
            b1 = 1 + 2 * t
            b2 = 2 + 2 * t
            drain_idx(1)
            fire_scat(1)
            drain_scat(0)

            @pl.when(b1 + 1 < iters)
            def _p1():
                fire_idx(b1 + 1, 0)
                drain_idx(0)

            @pl.when(b1 + 1 < iters)
            def _s2():
                fire_scat(0)
            drain_scat(1)

            @pl.when(b2 + 1 < iters)
            def _p2():
                fire_idx(b2 + 1, 1)

            return _

        lax.fori_loop(0, (iters - 1) // 2, pair, 0)
        drain_scat(0)
        plsc.subcore_barrier()
        pltpu.sync_copy(acc.at[pl.ds(s * slab, slab)],
                        out_hbm.at[c, pl.ds(s * slab, slab)])

    return k(col)


def _edge_scatter(hs, row, col):
    n, d = hs.shape
    e = row.shape[0]
    npad = -(-n // (NS * 128)) * (NS * 128)
    slab = npad // NS
    ew = e // (NC * NS)
    sub = _pick_chunk(ew)
    g_total = ew // sub
    t_body = (g_total - 3) // 6
    r_tail = g_total - 6 * t_body
    assert g_total >= 6, (ew, sub)
    zr = 8
    for cand in range(min(slab, 64), 0, -8):
        if slab % cand == 0:
            zr = cand
            break

    @functools.partial(
        pl.kernel,
        out_type=jax.ShapeDtypeStruct((NC, npad, d), jnp.float32),
        mesh=_sc_mesh(),
        scratch_types=[
            pltpu.VMEM((6, 2, sub), jnp.int32),
            pltpu.VMEM((3, sub, d), jnp.float32),
            pltpu.VMEM((zr, d), jnp.float32),
            pltpu.VMEM_SHARED((npad, d), jnp.float32),
            pltpu.SemaphoreType.DMA,
            pltpu.SemaphoreType.DMA,
            pltpu.SemaphoreType.DMA,
            pltpu.SemaphoreType.DMA,
            pltpu.SemaphoreType.DMA,
        ],
    )
    def k(hs_hbm, row_hbm, col_hbm, out_hbm, eidx, rows, zbuf, acc,
          sem_i, sem_g0, sem_g1, sem_g2, sem_s):
        c = lax.axis_index("c")
        s = lax.axis_index("s")
        w = c * NS + s
        base = w * ew
        sem_g = (sem_g0, sem_g1, sem_g2)

        def fire_eidx(g, ib):
            off = base + g * sub
            pltpu.async_copy(row_hbm.at[pl.ds(off, sub)], eidx.at[ib, 0], sem_i)
            pltpu.async_copy(col_hbm.at[pl.ds(off, sub)], eidx.at[ib, 1], sem_i)

        def drain_eidx(ib):
            pltpu.make_async_copy(row_hbm.at[pl.ds(base, sub)],
                                  eidx.at[ib, 0], sem_i).wait()
            pltpu.make_async_copy(col_hbm.at[pl.ds(base, sub)],
                                  eidx.at[ib, 1], sem_i).wait()

        def fire_gather(rb, ib):
            pltpu.async_copy(hs_hbm.at[eidx.at[ib, 0]], rows.at[rb], sem_g[rb])

        def drain_gather(rb, ib):
            pltpu.make_async_copy(hs_hbm.at[eidx.at[ib, 0]], rows.at[rb],
                                  sem_g[rb]).wait()

        def fire_scat(rb, ib):
            pltpu.async_copy(rows.at[rb], acc.at[eidx.at[ib, 1]], sem_s,
                             add=True)

        def drain_scat(rb, ib):
            pltpu.make_async_copy(rows.at[rb], acc.at[eidx.at[ib, 1]],
                                  sem_s).wait()

        for g in range(3):
            fire_eidx(g, g)

        def fill(i, _):
            for j in range(d // LANES):
                zbuf[i, pl.ds(j * LANES, LANES)] = jnp.zeros((LANES,), jnp.float32)
            return _

        lax.fori_loop(0, zr, fill, 0)
        for g in range(3):
            drain_eidx(g)
        for g in range(3):
            fire_gather(g, g)
        for t in range(slab // zr):
            pltpu.sync_copy(zbuf, acc.at[pl.ds(s * slab + t * zr, zr)])
        plsc.subcore_barrier()

        def slot(g, u):
            rb = u % 3
            ib = u % 6
            ib2 = (u + 3) % 6
            fire_eidx(g + 3, ib2)
            drain_gather(rb, ib)
            fire_scat(rb, ib)
            drain_scat(rb, ib)
            drain_eidx(ib2)
            fire_gather(rb, ib2)

        def body(t, _):
            g0 = 6 * t
            for u in range(6):
                slot(g0 + u, u)
            return _

        lax.fori_loop(0, t_body, body, 0)

        g0 = 6 * t_body
        for u in range(r_tail):
            rb = u % 3
            ib = u % 6
            if u + 3 < r_tail:
                fire_eidx(g0 + u + 3, (u + 3) % 6)
            drain_gather(rb, ib)
            fire_scat(rb, ib)
            drain_scat(rb, ib)
            if u + 3 < r_tail:
                drain_eidx((u + 3) % 6)
                fire_gather(rb, (u + 3) % 6)

        plsc.subcore_barrier()
        pltpu.sync_copy(acc.at[pl.ds(s * slab, slab)],
                        out_hbm.at[c, pl.ds(s * slab, slab)])

    return k(hs, row, col)


def _dis_col(deg_ref, n):
    deg = deg_ref[0, :n] + deg_ref[1, :n] + 1.0
    return lax.rsqrt(deg)[:, None]


def _mm_pre(x, W, degp):
    n, d = x.shape

    def body(x_ref, w_ref, deg_ref, o_ref):
        h = jnp.dot(x_ref[...], w_ref[...], preferred_element_type=jnp.float32)
        o_ref[...] = h * _dis_col(deg_ref, n)

    return pl.pallas_call(
        body, out_shape=jax.ShapeDtypeStruct((n, d), jnp.float32)
    )(x, W, degp)


def _mm_mid(s1, hs1, degp, W2, b1):
    n, d = hs1.shape

    def body(s1_ref, hs1_ref, deg_ref, w_ref, b_ref, o_ref):
        dis = _dis_col(deg_ref, n)
        pre = (dis * (s1_ref[0, :n] + s1_ref[1, :n] + hs1_ref[...])
               + b_ref[...][None, :])
        a = jnp.maximum(pre, 0.0)
        o_ref[...] = jnp.dot(a, w_ref[...], preferred_element_type=jnp.float32) * dis

    return pl.pallas_call(
        body, out_shape=jax.ShapeDtypeStruct((n, d), jnp.float32)
    )(s1, hs1, degp, W2, b1)


def _mm_post(s2, hs2, degp, b2):
    n, d = hs2.shape

    def body(s2_ref, hs2_ref, deg_ref, b_ref, o_ref):
        dis = _dis_col(deg_ref, n)
        o_ref[...] = (dis * (s2_ref[0, :n] + s2_ref[1, :n] + hs2_ref[...])
                      + b_ref[...][None, :])

    return pl.pallas_call(
        body, out_shape=jax.ShapeDtypeStruct((n, d), jnp.float32)
    )(s2, hs2, degp, b2)


def kernel(x, edge_index, W1, b1, W2, b2):
    n = x.shape[0]
    row = edge_index[0]
    col = edge_index[1]
    degp = _deg_histogram(col, n)
    hs1 = _mm_pre(x, W1, degp)
    s1 = _edge_scatter(hs1, row, col)
    hs2 = _mm_mid(s1, hs1, degp, W2, b1)
    s2 = _edge_scatter(hs2, row, col)
    return _mm_post(s2, hs2, degp, b2)
